# baseline (speedup 1.0000x reference)
import jax
import jax.numpy as jnp
from jax.experimental import pallas as pl
from jax.experimental.pallas import tpu as pltpu

B_, T_, D_ = 1, 2048, 2048
H_, HKV_, HD_ = 16, 4, 128
BS_, K_, W_ = 64, 16, 512
NB_ = T_ // BS_
G_ = H_ // HKV_
NEG_ = -1e30
SCALE_ = HD_ ** -0.5
QB_ = 256
NQ_ = T_ // QB_
PPAD_ = 3200


def _msoftmax(s, mask):
    s = jnp.where(mask, s, NEG_)
    m = jnp.max(s, axis=-1, keepdims=True)
    e = jnp.where(mask, jnp.exp(s - m), 0.0)
    d = jnp.sum(e, axis=-1, keepdims=True)
    return e / jnp.maximum(d, 1e-20)


def _proj_body(x_ref, w_ref, q_ref, k_ref, v_ref, g_ref):
    y = jax.lax.dot_general(x_ref[...], w_ref[...], (((1,), (0,)), ((), ())),
                            preferred_element_type=jnp.float32)
    q_ref[...] = y[:, :2048] * SCALE_
    k_ref[...] = y[:, 2048:2560]
    v_ref[...] = y[:, 2560:3072]
    g_ref[...] = y[:, 3072:3200]


def _pool_body(k_ref, v_ref, kb_ref, vb_ref):
    kb_ref[...] = jnp.mean(k_ref[...].reshape(NB_, BS_, HKV_ * HD_), axis=1)
    vb_ref[...] = jnp.mean(v_ref[...].reshape(NB_, BS_, HKV_ * HD_), axis=1)


def _cmp_body(q_ref, kb_ref, vb_ref, ocmp_ref, sel_ref):
    i = pl.program_id(0)
    t = i * QB_ + jax.lax.broadcasted_iota(jnp.int32, (QB_, NB_), 0)
    n = jax.lax.broadcasted_iota(jnp.int32, (QB_, NB_), 1)
    m_cmp = ((n + 1) * BS_ - 1) <= t
    force = (n == (t // BS_)) | (n == 0)
    jj = jax.lax.broadcasted_iota(jnp.int32, (NB_, NB_), 1)
    ii = jax.lax.broadcasted_iota(jnp.int32, (NB_, NB_), 0)
    tie = (jj < ii)[None]
    for hk in range(HKV_):
        kb = kb_ref[:, hk * HD_:(hk + 1) * HD_]
        vb = vb_ref[:, hk * HD_:(hk + 1) * HD_]
        imp = jnp.zeros((QB_, NB_), jnp.float32)
        for g in range(G_):
            h = hk * G_ + g
            qh = q_ref[:, h * HD_:(h + 1) * HD_]
            s = jax.lax.dot_general(qh, kb, (((1,), (1,)), ((), ())),
                                    preferred_element_type=jnp.float32)
            p = _msoftmax(s, m_cmp)
            o = jax.lax.dot_general(p, vb, (((1,), (0,)), ((), ())),
                                    preferred_element_type=jnp.float32)
            ocmp_ref[:, h * HD_:(h + 1) * HD_] = o
            imp = imp + p
        imp = imp + jnp.where(force, 1e9, 0.0)
        beats = (imp[:, None, :] > imp[:, :, None]) | (
            (imp[:, None, :] == imp[:, :, None]) & tie)
        rank = jnp.sum(beats.astype(jnp.float32), axis=-1)
        sel_ref[:, hk * NB_:(hk + 1) * NB_] = (rank < K_).astype(jnp.float32)


KB_ = 256
NJ_ = T_ // KB_
SWJ_ = (W_ + QB_ - 1) // KB_ + 1


def _attn_body(q_ref, k_ref, v_ref, g_ref, ocmp_ref, sel_ref, o_ref,
               acc_slc, acc_swa, m_slc_s, d_slc_s, m_swa_s, d_swa_s):
    i = pl.program_id(0)
    j = pl.program_id(1)

    @pl.when(j == 0)
    def _init():
        m_slc_s[...] = jnp.full((QB_, 128), NEG_, jnp.float32)
        m_swa_s[...] = jnp.full((QB_, 128), NEG_, jnp.float32)
        d_slc_s[...] = jnp.zeros((QB_, 128), jnp.float32)
        d_swa_s[...] = jnp.zeros((QB_, 128), jnp.float32)
        acc_slc[...] = jnp.zeros((QB_, H_ * HD_), jnp.float32)
        acc_swa[...] = jnp.zeros((QB_, H_ * HD_), jnp.float32)

    @pl.when(j <= i)
    def _work():
        t = i * QB_ + jax.lax.broadcasted_iota(jnp.int32, (QB_, KB_), 0)
        sc = j * KB_ + jax.lax.broadcasted_iota(jnp.int32, (QB_, KB_), 1)
        causal = sc <= t
        swa_mask = causal & (sc > t - W_)
        en = jax.lax.broadcasted_iota(jnp.int32, (NB_, KB_), 0)
        ec = j * KB_ + jax.lax.broadcasted_iota(jnp.int32, (NB_, KB_), 1)
        expand = ((ec // BS_) == en).astype(jnp.float32)
        swa_live = j >= i - (SWJ_ - 1)
        for hk in range(HKV_):
            kk = k_ref[:, hk * HD_:(hk + 1) * HD_]
            vv = v_ref[:, hk * HD_:(hk + 1) * HD_]
            selc = sel_ref[:, hk * NB_:(hk + 1) * NB_]
            selexp = jax.lax.dot_general(selc, expand, (((1,), (0,)), ((), ())),
                                         preferred_element_type=jnp.float32)
            slc_mask = (selexp > 0.5) & causal
            for g in range(G_):
                h = hk * G_ + g
                qh = q_ref[:, h * HD_:(h + 1) * HD_]
                s = jax.lax.dot_general(qh, kk, (((1,), (1,)), ((), ())),
                                        preferred_element_type=jnp.float32)
                for (msk, m_s, d_s, acc, live) in (
                        (slc_mask, m_slc_s, d_slc_s, acc_slc, True),
                        (swa_mask, m_swa_s, d_swa_s, acc_swa, swa_live)):
                    @pl.when(live)
                    def _upd(msk=msk, m_s=m_s, d_s=d_s, acc=acc):
                        sm = jnp.where(msk, s, NEG_)
                        m_old = m_s[:, h:h + 1]
                        m_new = jnp.maximum(m_old, jnp.max(sm, axis=-1, keepdims=True))
                        scale = jnp.exp(m_old - m_new)
                        e = jnp.where(msk, jnp.exp(sm - m_new), 0.0)
                        pv = jax.lax.dot_general(e, vv, (((1,), (0,)), ((), ())),
                                                 preferred_element_type=jnp.float32)
                        m_s[:, h:h + 1] = m_new
                        d_s[:, h:h + 1] = d_s[:, h:h + 1] * scale + jnp.sum(
                            e, axis=-1, keepdims=True)
                        acc[:, h * HD_:(h + 1) * HD_] = (
                            acc[:, h * HD_:(h + 1) * HD_] * scale + pv)

    @pl.when(j == i)
    def _finish():
        gates = jax.nn.sigmoid(g_ref[:, :H_ * 3])
        for h in range(H_):
            cols = slice(h * HD_, (h + 1) * HD_)
            o_slc = acc_slc[:, cols] / jnp.maximum(d_slc_s[:, h:h + 1], 1e-20)
            o_swa = acc_swa[:, cols] / jnp.maximum(d_swa_s[:, h:h + 1], 1e-20)
            gc = gates[:, 3 * h:3 * h + 1]
            gs = gates[:, 3 * h + 1:3 * h + 2]
            gw = gates[:, 3 * h + 2:3 * h + 3]
            o_ref[:, cols] = ocmp_ref[:, cols] * gc + o_slc * gs + o_swa * gw


def _oproj_body(z_ref, w_ref, o_ref):
    o_ref[...] = jax.lax.dot_general(z_ref[...], w_ref[...], (((1,), (0,)), ((), ())),
                                     preferred_element_type=jnp.float32)


def _nsa_pallas(x, WcatT, WoT, interpret=False):
    f32 = jnp.float32
    q, k, v, g = pl.pallas_call(
        _proj_body,
        grid=(NQ_,),
        in_specs=[
            pl.BlockSpec((QB_, D_), lambda i: (i, 0)),
            pl.BlockSpec((D_, PPAD_), lambda i: (0, 0)),
        ],
        out_specs=[
            pl.BlockSpec((QB_, 2048), lambda i: (i, 0)),
            pl.BlockSpec((QB_, 512), lambda i: (i, 0)),
            pl.BlockSpec((QB_, 512), lambda i: (i, 0)),
            pl.BlockSpec((QB_, 128), lambda i: (i, 0)),
        ],
        out_shape=[
            jax.ShapeDtypeStruct((T_, 2048), f32),
            jax.ShapeDtypeStruct((T_, 512), f32),
            jax.ShapeDtypeStruct((T_, 512), f32),
            jax.ShapeDtypeStruct((T_, 128), f32),
        ],
        interpret=interpret,
    )(x, WcatT)

    kb, vb = pl.pallas_call(
        _pool_body,
        out_shape=[
            jax.ShapeDtypeStruct((NB_, HKV_ * HD_), f32),
            jax.ShapeDtypeStruct((NB_, HKV_ * HD_), f32),
        ],
        interpret=interpret,
    )(k, v)

    ocmp, sel = pl.pallas_call(
        _cmp_body,
        grid=(NQ_,),
        in_specs=[
            pl.BlockSpec((QB_, 2048), lambda i: (i, 0)),
            pl.BlockSpec((NB_, HKV_ * HD_), lambda i: (0, 0)),
            pl.BlockSpec((NB_, HKV_ * HD_), lambda i: (0, 0)),
        ],
        out_specs=[
            pl.BlockSpec((QB_, 2048), lambda i: (i, 0)),
            pl.BlockSpec((QB_, HKV_ * NB_), lambda i: (i, 0)),
        ],
        out_shape=[
            jax.ShapeDtypeStruct((T_, 2048), f32),
            jax.ShapeDtypeStruct((T_, HKV_ * NB_), f32),
        ],
        interpret=interpret,
    )(q, kb, vb)

    z = pl.pallas_call(
        _attn_body,
        grid=(NQ_, NJ_),
        in_specs=[
            pl.BlockSpec((QB_, 2048), lambda i, j: (i, 0)),
            pl.BlockSpec((KB_, 512), lambda i, j: (j, 0)),
            pl.BlockSpec((KB_, 512), lambda i, j: (j, 0)),
            pl.BlockSpec((QB_, 128), lambda i, j: (i, 0)),
            pl.BlockSpec((QB_, 2048), lambda i, j: (i, 0)),
            pl.BlockSpec((QB_, HKV_ * NB_), lambda i, j: (i, 0)),
        ],
        out_specs=pl.BlockSpec((QB_, 2048), lambda i, j: (i, 0)),
        out_shape=jax.ShapeDtypeStruct((T_, 2048), f32),
        scratch_shapes=[
            pltpu.VMEM((QB_, H_ * HD_), f32),
            pltpu.VMEM((QB_, H_ * HD_), f32),
            pltpu.VMEM((QB_, 128), f32),
            pltpu.VMEM((QB_, 128), f32),
            pltpu.VMEM((QB_, 128), f32),
            pltpu.VMEM((QB_, 128), f32),
        ],
        interpret=interpret,
    )(q, k, v, g, ocmp, sel)

    out = pl.pallas_call(
        _oproj_body,
        grid=(NQ_,),
        in_specs=[
            pl.BlockSpec((QB_, 2048), lambda i: (i, 0)),
            pl.BlockSpec((D_, D_), lambda i: (0, 0)),
        ],
        out_specs=pl.BlockSpec((QB_, D_), lambda i: (i, 0)),
        out_shape=jax.ShapeDtypeStruct((T_, D_), f32),
        interpret=interpret,
    )(z, WoT)
    return out


def kernel(hidden_states, Wq, Wk, Wv, Wg, Wo):
    x = hidden_states.reshape(T_, D_)
    Wcat = jnp.concatenate([Wq, Wk, Wv,
                            jnp.pad(Wg, ((0, PPAD_ - 3072 - H_ * 3), (0, 0)))], axis=0)
    out = _nsa_pallas(x, Wcat.T, Wo.T)
    return out.reshape(B_, T_, D_)


# speedup vs baseline: 1.0173x; 1.0173x over previous
import jax
import jax.numpy as jnp
from jax.experimental import pallas as pl
from jax.experimental.pallas import tpu as pltpu

B_, T_, D_ = 1, 2048, 2048
H_, HKV_, HD_ = 16, 4, 128
BS_, K_, W_ = 64, 16, 512
NB_ = T_ // BS_
G_ = H_ // HKV_
NEG_ = -1e30
SCALE_ = HD_ ** -0.5
QB_ = 256
NQ_ = T_ // QB_
PPAD_ = 3200


def _msoftmax(s, mask):
    s = jnp.where(mask, s, NEG_)
    m = jnp.max(s, axis=-1, keepdims=True)
    e = jnp.where(mask, jnp.exp(s - m), 0.0)
    d = jnp.sum(e, axis=-1, keepdims=True)
    return e / jnp.maximum(d, 1e-20)


def _proj_body(x_ref, w_ref, q_ref, k_ref, v_ref, g_ref):
    y = jax.lax.dot_general(x_ref[...], w_ref[...], (((1,), (0,)), ((), ())),
                            preferred_element_type=jnp.float32)
    q_ref[...] = y[:, :2048] * SCALE_
    k_ref[...] = y[:, 2048:2560]
    v_ref[...] = y[:, 2560:3072]
    g_ref[...] = y[:, 3072:3200]


def _pool_body(k_ref, v_ref, kb_ref, vb_ref):
    kb_ref[...] = jnp.mean(k_ref[...].reshape(NB_, BS_, HKV_ * HD_), axis=1)
    vb_ref[...] = jnp.mean(v_ref[...].reshape(NB_, BS_, HKV_ * HD_), axis=1)


def _cmp_body(q_ref, kb_ref, vb_ref, ocmp_ref, sel_ref):
    i = pl.program_id(0)
    t = i * QB_ + jax.lax.broadcasted_iota(jnp.int32, (QB_, NB_), 0)
    n = jax.lax.broadcasted_iota(jnp.int32, (QB_, NB_), 1)
    m_cmp = ((n + 1) * BS_ - 1) <= t
    force = (n == (t // BS_)) | (n == 0)
    jj = jax.lax.broadcasted_iota(jnp.int32, (NB_, NB_), 1)
    ii = jax.lax.broadcasted_iota(jnp.int32, (NB_, NB_), 0)
    tie = (jj < ii)[None]
    for hk in range(HKV_):
        kb = kb_ref[:, hk * HD_:(hk + 1) * HD_]
        vb = vb_ref[:, hk * HD_:(hk + 1) * HD_]
        imp = jnp.zeros((QB_, NB_), jnp.float32)
        for g in range(G_):
            h = hk * G_ + g
            qh = q_ref[:, h * HD_:(h + 1) * HD_]
            s = jax.lax.dot_general(qh, kb, (((1,), (1,)), ((), ())),
                                    preferred_element_type=jnp.float32)
            p = _msoftmax(s, m_cmp)
            o = jax.lax.dot_general(p, vb, (((1,), (0,)), ((), ())),
                                    preferred_element_type=jnp.float32)
            ocmp_ref[:, h * HD_:(h + 1) * HD_] = o
            imp = imp + p
        imp = imp + jnp.where(force, 1e9, 0.0)
        beats = (imp[:, None, :] > imp[:, :, None]) | (
            (imp[:, None, :] == imp[:, :, None]) & tie)
        rank = jnp.sum(beats.astype(jnp.float32), axis=-1)
        sel_ref[:, hk * NB_:(hk + 1) * NB_] = (rank < K_).astype(jnp.float32)


KB_ = 256
NJ_ = T_ // KB_
SWJ_ = (W_ + QB_ - 1) // KB_


def _attn_body(q_ref, k_ref, v_ref, g_ref, ocmp_ref, sel_ref, o_ref, s_scr):
    i = pl.program_id(0)
    gates = jax.nn.sigmoid(g_ref[:, :H_ * 3])
    row = jax.lax.broadcasted_iota(jnp.int32, (QB_, KB_), 0)
    col = jax.lax.broadcasted_iota(jnp.int32, (QB_, KB_), 1)
    en = jax.lax.broadcasted_iota(jnp.int32, (NB_, KB_), 0)
    ec = jax.lax.broadcasted_iota(jnp.int32, (NB_, KB_), 1)
    for hk in range(HKV_):
        hcols = slice(hk * HD_, (hk + 1) * HD_)
        selc = sel_ref[:, hk * NB_:(hk + 1) * NB_]
        for g in range(G_):
            h = hk * G_ + g
            qh = q_ref[:, h * HD_:(h + 1) * HD_]

            def pass_a(j, m, qh=qh, hcols=hcols):
                kk = k_ref[pl.ds(j * KB_, KB_), hcols]
                s = jax.lax.dot_general(qh, kk, (((1,), (1,)), ((), ())),
                                        preferred_element_type=jnp.float32)
                causal = (j * KB_ + col) <= (i * QB_ + row)
                sm = jnp.where(causal, s, NEG_)
                s_scr[j] = sm
                return jnp.maximum(m, jnp.max(sm, axis=-1, keepdims=True))

            m = jax.lax.fori_loop(0, i + 1, pass_a,
                                  jnp.full((QB_, 1), NEG_, jnp.float32))

            def pass_slc(j, carry, selc=selc, hcols=hcols, m=m):
                num, den = carry
                es = jnp.exp(s_scr[j] - m)
                expand = (((j * KB_ + ec) // BS_) == en).astype(jnp.float32)
                selx = jax.lax.dot_general(selc, expand, (((1,), (0,)), ((), ())),
                                           preferred_element_type=jnp.float32)
                e = es * selx
                vv = v_ref[pl.ds(j * KB_, KB_), hcols]
                num = num + jax.lax.dot_general(e, vv, (((1,), (0,)), ((), ())),
                                                preferred_element_type=jnp.float32)
                return num, den + jnp.sum(e, axis=-1, keepdims=True)

            num_slc, d_slc = jax.lax.fori_loop(
                0, i + 1, pass_slc,
                (jnp.zeros((QB_, HD_), jnp.float32), jnp.zeros((QB_, 1), jnp.float32)))

            def pass_swa(j, carry, hcols=hcols, m=m):
                num, den = carry
                es = jnp.exp(s_scr[j] - m)
                t_idx = i * QB_ + row
                c_idx = j * KB_ + col
                e = jnp.where(c_idx > t_idx - W_, es, 0.0)
                vv = v_ref[pl.ds(j * KB_, KB_), hcols]
                num = num + jax.lax.dot_general(e, vv, (((1,), (0,)), ((), ())),
                                                preferred_element_type=jnp.float32)
                return num, den + jnp.sum(e, axis=-1, keepdims=True)

            num_swa, d_swa = jax.lax.fori_loop(
                jnp.maximum(0, i - SWJ_), i + 1, pass_swa,
                (jnp.zeros((QB_, HD_), jnp.float32), jnp.zeros((QB_, 1), jnp.float32)))

            o_slc = num_slc / jnp.maximum(d_slc, 1e-20)
            o_swa = num_swa / jnp.maximum(d_swa, 1e-20)
            cols = slice(h * HD_, (h + 1) * HD_)
            gc = gates[:, 3 * h:3 * h + 1]
            gs = gates[:, 3 * h + 1:3 * h + 2]
            gw = gates[:, 3 * h + 2:3 * h + 3]
            o_ref[:, cols] = ocmp_ref[:, cols] * gc + o_slc * gs + o_swa * gw


def _oproj_body(z_ref, w_ref, o_ref):
    o_ref[...] = jax.lax.dot_general(z_ref[...], w_ref[...], (((1,), (0,)), ((), ())),
                                     preferred_element_type=jnp.float32)


def _nsa_pallas(x, WcatT, WoT, interpret=False):
    f32 = jnp.float32
    q, k, v, g = pl.pallas_call(
        _proj_body,
        grid=(NQ_,),
        in_specs=[
            pl.BlockSpec((QB_, D_), lambda i: (i, 0)),
            pl.BlockSpec((D_, PPAD_), lambda i: (0, 0)),
        ],
        out_specs=[
            pl.BlockSpec((QB_, 2048), lambda i: (i, 0)),
            pl.BlockSpec((QB_, 512), lambda i: (i, 0)),
            pl.BlockSpec((QB_, 512), lambda i: (i, 0)),
            pl.BlockSpec((QB_, 128), lambda i: (i, 0)),
        ],
        out_shape=[
            jax.ShapeDtypeStruct((T_, 2048), f32),
            jax.ShapeDtypeStruct((T_, 512), f32),
            jax.ShapeDtypeStruct((T_, 512), f32),
            jax.ShapeDtypeStruct((T_, 128), f32),
        ],
        interpret=interpret,
    )(x, WcatT)

    kb, vb = pl.pallas_call(
        _pool_body,
        out_shape=[
            jax.ShapeDtypeStruct((NB_, HKV_ * HD_), f32),
            jax.ShapeDtypeStruct((NB_, HKV_ * HD_), f32),
        ],
        interpret=interpret,
    )(k, v)

    ocmp, sel = pl.pallas_call(
        _cmp_body,
        grid=(NQ_,),
        in_specs=[
            pl.BlockSpec((QB_, 2048), lambda i: (i, 0)),
            pl.BlockSpec((NB_, HKV_ * HD_), lambda i: (0, 0)),
            pl.BlockSpec((NB_, HKV_ * HD_), lambda i: (0, 0)),
        ],
        out_specs=[
            pl.BlockSpec((QB_, 2048), lambda i: (i, 0)),
            pl.BlockSpec((QB_, HKV_ * NB_), lambda i: (i, 0)),
        ],
        out_shape=[
            jax.ShapeDtypeStruct((T_, 2048), f32),
            jax.ShapeDtypeStruct((T_, HKV_ * NB_), f32),
        ],
        interpret=interpret,
    )(q, kb, vb)

    z = pl.pallas_call(
        _attn_body,
        grid=(NQ_,),
        in_specs=[
            pl.BlockSpec((QB_, 2048), lambda i: (i, 0)),
            pl.BlockSpec((T_, 512), lambda i: (0, 0)),
            pl.BlockSpec((T_, 512), lambda i: (0, 0)),
            pl.BlockSpec((QB_, 128), lambda i: (i, 0)),
            pl.BlockSpec((QB_, 2048), lambda i: (i, 0)),
            pl.BlockSpec((QB_, HKV_ * NB_), lambda i: (i, 0)),
        ],
        out_specs=pl.BlockSpec((QB_, 2048), lambda i: (i, 0)),
        out_shape=jax.ShapeDtypeStruct((T_, 2048), f32),
        scratch_shapes=[
            pltpu.VMEM((NJ_, QB_, KB_), f32),
        ],
        interpret=interpret,
    )(q, k, v, g, ocmp, sel)

    out = pl.pallas_call(
        _oproj_body,
        grid=(NQ_,),
        in_specs=[
            pl.BlockSpec((QB_, 2048), lambda i: (i, 0)),
            pl.BlockSpec((D_, D_), lambda i: (0, 0)),
        ],
        out_specs=pl.BlockSpec((QB_, D_), lambda i: (i, 0)),
        out_shape=jax.ShapeDtypeStruct((T_, D_), f32),
        interpret=interpret,
    )(z, WoT)
    return out


def kernel(hidden_states, Wq, Wk, Wv, Wg, Wo):
    x = hidden_states.reshape(T_, D_)
    Wcat = jnp.concatenate([Wq, Wk, Wv,
                            jnp.pad(Wg, ((0, PPAD_ - 3072 - H_ * 3), (0, 0)))], axis=0)
    out = _nsa_pallas(x, Wcat.T, Wo.T)
    return out.reshape(B_, T_, D_)


# speedup vs baseline: 1.7368x; 1.7072x over previous
import jax
import jax.numpy as jnp
from jax.experimental import pallas as pl
from jax.experimental.pallas import tpu as pltpu

B_, T_, D_ = 1, 2048, 2048
H_, HKV_, HD_ = 16, 4, 128
BS_, K_, W_ = 64, 16, 512
NB_ = T_ // BS_
G_ = H_ // HKV_
NEG_ = -1e30
SCALE_ = HD_ ** -0.5
QB_ = 256
NQ_ = T_ // QB_
PPAD_ = 3200


def _msoftmax(s, mask):
    s = jnp.where(mask, s, NEG_)
    m = jnp.max(s, axis=-1, keepdims=True)
    e = jnp.where(mask, jnp.exp(s - m), 0.0)
    d = jnp.sum(e, axis=-1, keepdims=True)
    return e / jnp.maximum(d, 1e-20)


def _proj_body(x_ref, w_ref, q_ref, k_ref, v_ref, g_ref):
    y = jax.lax.dot_general(x_ref[...], w_ref[...], (((1,), (0,)), ((), ())),
                            preferred_element_type=jnp.float32)
    for h in range(H_):
        q_ref[h] = y[:, h * HD_:(h + 1) * HD_] * SCALE_
    k_ref[...] = y[:, 2048:2560]
    v_ref[...] = y[:, 2560:3072]
    g_ref[...] = y[:, 3072:3200]


def _pool_body(k_ref, v_ref, kb_ref, vb_ref):
    kb_ref[...] = jnp.mean(k_ref[...].reshape(NB_, BS_, HKV_ * HD_), axis=1)
    vb_ref[...] = jnp.mean(v_ref[...].reshape(NB_, BS_, HKV_ * HD_), axis=1)


def _cmp_body(q_ref, kb_ref, vb_ref, ocmp_ref, sel_ref):
    i = pl.program_id(0)
    r4 = jax.lax.broadcasted_iota(jnp.int32, (G_ * QB_, NB_), 0)
    t4 = i * QB_ + r4 % QB_
    n4 = jax.lax.broadcasted_iota(jnp.int32, (G_ * QB_, NB_), 1)
    m_cmp4 = ((n4 + 1) * BS_ - 1) <= t4
    t = i * QB_ + jax.lax.broadcasted_iota(jnp.int32, (QB_, NB_), 0)
    n = jax.lax.broadcasted_iota(jnp.int32, (QB_, NB_), 1)
    force = (n == (t // BS_)) | (n == 0)
    jj = jax.lax.broadcasted_iota(jnp.int32, (NB_, NB_), 1)
    ii = jax.lax.broadcasted_iota(jnp.int32, (NB_, NB_), 0)
    tie = (jj < ii)[None]
    for hk in range(HKV_):
        kb = kb_ref[:, hk * HD_:(hk + 1) * HD_]
        vb = vb_ref[:, hk * HD_:(hk + 1) * HD_]
        qg = q_ref[hk * G_:(hk + 1) * G_].reshape(G_ * QB_, HD_)
        s = jax.lax.dot_general(qg, kb, (((1,), (1,)), ((), ())),
                                preferred_element_type=jnp.float32)
        p = _msoftmax(s, m_cmp4)
        o = jax.lax.dot_general(p, vb, (((1,), (0,)), ((), ())),
                                preferred_element_type=jnp.float32)
        for g in range(G_):
            h = hk * G_ + g
            ocmp_ref[:, h * HD_:(h + 1) * HD_] = o[g * QB_:(g + 1) * QB_]
        imp = jnp.sum(p.reshape(G_, QB_, NB_), axis=0)
        imp = imp + jnp.where(force, 1e9, 0.0)
        beats = (imp[:, None, :] > imp[:, :, None]) | (
            (imp[:, None, :] == imp[:, :, None]) & tie)
        rank = jnp.sum(beats.astype(jnp.float32), axis=-1)
        sel_ref[:, hk * NB_:(hk + 1) * NB_] = (rank < K_).astype(jnp.float32)


def _attn_body(q_ref, k_ref, v_ref, g_ref, ocmp_ref, sel_ref, o_ref):
    i = pl.program_id(0)
    gates = jax.nn.sigmoid(g_ref[:, :H_ * 3])
    r4 = jax.lax.broadcasted_iota(jnp.int32, (G_ * QB_, T_), 0)
    t4 = i * QB_ + r4 % QB_
    c4 = jax.lax.broadcasted_iota(jnp.int32, (G_ * QB_, T_), 1)
    causal4 = c4 <= t4
    swa4 = c4 > t4 - W_
    en = jax.lax.broadcasted_iota(jnp.int32, (NB_, T_), 0)
    es_ = jax.lax.broadcasted_iota(jnp.int32, (NB_, T_), 1)
    expand = ((es_ // BS_) == en).astype(jnp.float32)
    for hk in range(HKV_):
        kk = k_ref[:, hk * HD_:(hk + 1) * HD_]
        vv = v_ref[:, hk * HD_:(hk + 1) * HD_]
        selc = sel_ref[:, hk * NB_:(hk + 1) * NB_]
        selx = jax.lax.dot_general(selc, expand, (((1,), (0,)), ((), ())),
                                   preferred_element_type=jnp.float32)
        sel4 = jnp.broadcast_to((selx > 0.5)[None], (G_, QB_, T_)).reshape(
            G_ * QB_, T_)
        qg = q_ref[hk * G_:(hk + 1) * G_].reshape(G_ * QB_, HD_)
        s = jax.lax.dot_general(qg, kk, (((1,), (1,)), ((), ())),
                                preferred_element_type=jnp.float32)
        sm = jnp.where(causal4, s, NEG_)
        m = jnp.max(sm, axis=-1, keepdims=True)
        es = jnp.exp(sm - m)
        e_slc = jnp.where(sel4, es, 0.0)
        e_swa = jnp.where(swa4, es, 0.0)
        num_slc = jax.lax.dot_general(e_slc, vv, (((1,), (0,)), ((), ())),
                                      preferred_element_type=jnp.float32)
        num_swa = jax.lax.dot_general(e_swa, vv, (((1,), (0,)), ((), ())),
                                      preferred_element_type=jnp.float32)
        d_slc = jnp.sum(e_slc, axis=-1, keepdims=True)
        d_swa = jnp.sum(e_swa, axis=-1, keepdims=True)
        o_slc = num_slc / jnp.maximum(d_slc, 1e-20)
        o_swa = num_swa / jnp.maximum(d_swa, 1e-20)
        for g in range(G_):
            h = hk * G_ + g
            rows = slice(g * QB_, (g + 1) * QB_)
            cols = slice(h * HD_, (h + 1) * HD_)
            gc = gates[:, 3 * h:3 * h + 1]
            gs = gates[:, 3 * h + 1:3 * h + 2]
            gw = gates[:, 3 * h + 2:3 * h + 3]
            o_ref[:, cols] = (ocmp_ref[:, cols] * gc + o_slc[rows] * gs
                              + o_swa[rows] * gw)


def _oproj_body(z_ref, w_ref, o_ref):
    o_ref[...] = jax.lax.dot_general(z_ref[...], w_ref[...], (((1,), (0,)), ((), ())),
                                     preferred_element_type=jnp.float32)


def _nsa_pallas(x, WcatT, WoT, interpret=False):
    f32 = jnp.float32
    q, k, v, g = pl.pallas_call(
        _proj_body,
        grid=(NQ_,),
        in_specs=[
            pl.BlockSpec((QB_, D_), lambda i: (i, 0)),
            pl.BlockSpec((D_, PPAD_), lambda i: (0, 0)),
        ],
        out_specs=[
            pl.BlockSpec((H_, QB_, HD_), lambda i: (0, i, 0)),
            pl.BlockSpec((QB_, 512), lambda i: (i, 0)),
            pl.BlockSpec((QB_, 512), lambda i: (i, 0)),
            pl.BlockSpec((QB_, 128), lambda i: (i, 0)),
        ],
        out_shape=[
            jax.ShapeDtypeStruct((H_, T_, HD_), f32),
            jax.ShapeDtypeStruct((T_, 512), f32),
            jax.ShapeDtypeStruct((T_, 512), f32),
            jax.ShapeDtypeStruct((T_, 128), f32),
        ],
        interpret=interpret,
    )(x, WcatT)

    kb, vb = pl.pallas_call(
        _pool_body,
        out_shape=[
            jax.ShapeDtypeStruct((NB_, HKV_ * HD_), f32),
            jax.ShapeDtypeStruct((NB_, HKV_ * HD_), f32),
        ],
        interpret=interpret,
    )(k, v)

    ocmp, sel = pl.pallas_call(
        _cmp_body,
        grid=(NQ_,),
        in_specs=[
            pl.BlockSpec((H_, QB_, HD_), lambda i: (0, i, 0)),
            pl.BlockSpec((NB_, HKV_ * HD_), lambda i: (0, 0)),
            pl.BlockSpec((NB_, HKV_ * HD_), lambda i: (0, 0)),
        ],
        out_specs=[
            pl.BlockSpec((QB_, 2048), lambda i: (i, 0)),
            pl.BlockSpec((QB_, HKV_ * NB_), lambda i: (i, 0)),
        ],
        out_shape=[
            jax.ShapeDtypeStruct((T_, 2048), f32),
            jax.ShapeDtypeStruct((T_, HKV_ * NB_), f32),
        ],
        interpret=interpret,
    )(q, kb, vb)

    z = pl.pallas_call(
        _attn_body,
        grid=(NQ_,),
        in_specs=[
            pl.BlockSpec((H_, QB_, HD_), lambda i: (0, i, 0)),
            pl.BlockSpec((T_, 512), lambda i: (0, 0)),
            pl.BlockSpec((T_, 512), lambda i: (0, 0)),
            pl.BlockSpec((QB_, 128), lambda i: (i, 0)),
            pl.BlockSpec((QB_, 2048), lambda i: (i, 0)),
            pl.BlockSpec((QB_, HKV_ * NB_), lambda i: (i, 0)),
        ],
        out_specs=pl.BlockSpec((QB_, 2048), lambda i: (i, 0)),
        out_shape=jax.ShapeDtypeStruct((T_, 2048), f32),
        interpret=interpret,
    )(q, k, v, g, ocmp, sel)

    out = pl.pallas_call(
        _oproj_body,
        grid=(NQ_,),
        in_specs=[
            pl.BlockSpec((QB_, 2048), lambda i: (i, 0)),
            pl.BlockSpec((D_, D_), lambda i: (0, 0)),
        ],
        out_specs=pl.BlockSpec((QB_, D_), lambda i: (i, 0)),
        out_shape=jax.ShapeDtypeStruct((T_, D_), f32),
        interpret=interpret,
    )(z, WoT)
    return out


def kernel(hidden_states, Wq, Wk, Wv, Wg, Wo):
    x = hidden_states.reshape(T_, D_)
    Wcat = jnp.concatenate([Wq, Wk, Wv,
                            jnp.pad(Wg, ((0, PPAD_ - 3072 - H_ * 3), (0, 0)))], axis=0)
    out = _nsa_pallas(x, Wcat.T, Wo.T)
    return out.reshape(B_, T_, D_)


# speedup vs baseline: 2.7081x; 1.5592x over previous
import jax
import jax.numpy as jnp
from jax.experimental import pallas as pl
from jax.experimental.pallas import tpu as pltpu

B_, T_, D_ = 1, 2048, 2048
H_, HKV_, HD_ = 16, 4, 128
BS_, K_, W_ = 64, 16, 512
NB_ = T_ // BS_
G_ = H_ // HKV_
NEG_ = -1e30
SCALE_ = HD_ ** -0.5
QB_ = 256
NQ_ = T_ // QB_
PPAD_ = 3200


def _msoftmax(s, mask):
    s = jnp.where(mask, s, NEG_)
    m = jnp.max(s, axis=-1, keepdims=True)
    e = jnp.where(mask, jnp.exp(s - m), 0.0)
    d = jnp.sum(e, axis=-1, keepdims=True)
    return e / jnp.maximum(d, 1e-20)


def _proj_body(x_ref, w_ref, q_ref, k_ref, v_ref, g_ref, vaug_ref):
    y = jax.lax.dot_general(x_ref[...], w_ref[...], (((1,), (0,)), ((), ())),
                            preferred_element_type=jnp.float32)
    for h in range(H_):
        q_ref[h] = y[:, h * HD_:(h + 1) * HD_] * SCALE_
    k_ref[...] = y[:, 2048:2560]
    v_ref[...] = y[:, 2560:3072]
    g_ref[...] = y[:, 3072:3200]
    for hk in range(HKV_):
        vaug_ref[:, hk * 256:hk * 256 + HD_] = (
            y[:, 2560 + hk * HD_:2560 + (hk + 1) * HD_].astype(jnp.bfloat16))
        vaug_ref[:, hk * 256 + HD_:(hk + 1) * 256] = jnp.ones(
            (QB_, 128), jnp.bfloat16)


def _pool_body(k_ref, v_ref, kb_ref, vb_ref):
    kb_ref[...] = jnp.mean(k_ref[...].reshape(NB_, BS_, HKV_ * HD_), axis=1)
    vb_ref[...] = jnp.mean(v_ref[...].reshape(NB_, BS_, HKV_ * HD_), axis=1)


def _cmp_body(q_ref, kb_ref, vb_ref, ocmp_ref, sel_ref):
    i = pl.program_id(0)
    nr = jax.lax.broadcasted_iota(jnp.int32, (NB_, G_ * QB_), 0)
    rc = jax.lax.broadcasted_iota(jnp.int32, (NB_, G_ * QB_), 1)
    t4 = i * QB_ + rc % QB_
    m_cmpT = ((nr + 1) * BS_ - 1) <= t4
    n1 = jax.lax.broadcasted_iota(jnp.int32, (NB_, QB_), 0)
    tl = i * QB_ + jax.lax.broadcasted_iota(jnp.int32, (NB_, QB_), 1)
    forceT = (n1 == (tl // BS_)) | (n1 == 0)
    force_add = jnp.where(forceT, 1e9, 0.0)
    for hk in range(HKV_):
        kb = kb_ref[:, hk * HD_:(hk + 1) * HD_]
        vb = vb_ref[:, hk * HD_:(hk + 1) * HD_]
        qg = q_ref[hk * G_:(hk + 1) * G_].reshape(G_ * QB_, HD_)
        sT = jax.lax.dot_general(kb, qg, (((1,), (1,)), ((), ())),
                                 preferred_element_type=jnp.float32)
        e = jnp.exp(jnp.where(m_cmpT, sT, NEG_))
        d = jnp.sum(e, axis=0, keepdims=True)
        pT = e / jnp.maximum(d, 1e-20)
        o = jax.lax.dot_general(pT, vb, (((0,), (0,)), ((), ())),
                                preferred_element_type=jnp.float32)
        for g in range(G_):
            h = hk * G_ + g
            ocmp_ref[:, h * HD_:(h + 1) * HD_] = o[g * QB_:(g + 1) * QB_]
        impT = jnp.sum(pT.reshape(NB_, G_, QB_), axis=1) + force_add
        cnt = jnp.zeros((NB_, QB_), jnp.float32)
        for mrow in range(NB_):
            vm = impT[mrow:mrow + 1, :]
            beats = (vm > impT) | ((vm == impT) & (mrow < n1))
            cnt = cnt + beats.astype(jnp.float32)
        selT = (cnt < K_).astype(jnp.float32)
        sel_ref[:, hk * NB_:(hk + 1) * NB_] = selT.T


def _attn_body(q_ref, k_ref, g_ref, ocmp_ref, sel_ref, vaug_ref, o_ref):
    i = pl.program_id(0)
    gates = jax.nn.sigmoid(g_ref[:, :H_ * 3])
    r4 = jax.lax.broadcasted_iota(jnp.int32, (G_ * QB_, T_), 0)
    t4 = i * QB_ + r4 % QB_
    c4 = jax.lax.broadcasted_iota(jnp.int32, (G_ * QB_, T_), 1)
    causal4 = c4 <= t4
    swa4 = c4 > t4 - W_
    en = jax.lax.broadcasted_iota(jnp.int32, (NB_, T_), 0)
    es_ = jax.lax.broadcasted_iota(jnp.int32, (NB_, T_), 1)
    expand = ((es_ // BS_) == en).astype(jnp.float32)
    for hk in range(HKV_):
        kk = k_ref[:, hk * HD_:(hk + 1) * HD_]
        vvaug = vaug_ref[:, hk * 256:(hk + 1) * 256]
        selc = sel_ref[:, hk * NB_:(hk + 1) * NB_]
        selx = jax.lax.dot_general(selc, expand, (((1,), (0,)), ((), ())),
                                   preferred_element_type=jnp.float32)
        sel4 = jnp.broadcast_to((selx > 0.5)[None], (G_, QB_, T_)).reshape(
            G_ * QB_, T_)
        qg = q_ref[hk * G_:(hk + 1) * G_].reshape(G_ * QB_, HD_)
        s = jax.lax.dot_general(qg, kk, (((1,), (1,)), ((), ())),
                                preferred_element_type=jnp.float32)
        es = jnp.exp(jnp.where(causal4, s, NEG_))
        e_slc = jnp.where(sel4, es, 0.0).astype(jnp.bfloat16)
        e_swa = jnp.where(swa4, es, 0.0).astype(jnp.bfloat16)
        nd_slc = jax.lax.dot_general(e_slc, vvaug, (((1,), (0,)), ((), ())),
                                     preferred_element_type=jnp.float32)
        nd_swa = jax.lax.dot_general(e_swa, vvaug, (((1,), (0,)), ((), ())),
                                     preferred_element_type=jnp.float32)
        o_slc = nd_slc[:, :HD_] / jnp.maximum(nd_slc[:, HD_:HD_ + 1], 1e-20)
        o_swa = nd_swa[:, :HD_] / jnp.maximum(nd_swa[:, HD_:HD_ + 1], 1e-20)
        for g in range(G_):
            h = hk * G_ + g
            rows = slice(g * QB_, (g + 1) * QB_)
            cols = slice(h * HD_, (h + 1) * HD_)
            gc = gates[:, 3 * h:3 * h + 1]
            gs = gates[:, 3 * h + 1:3 * h + 2]
            gw = gates[:, 3 * h + 2:3 * h + 3]
            o_ref[:, cols] = (ocmp_ref[:, cols] * gc + o_slc[rows] * gs
                              + o_swa[rows] * gw)


def _oproj_body(z_ref, w_ref, o_ref):
    o_ref[...] = jax.lax.dot_general(z_ref[...], w_ref[...], (((1,), (0,)), ((), ())),
                                     preferred_element_type=jnp.float32)


def _nsa_pallas(x, WcatT, WoT, interpret=False):
    f32 = jnp.float32
    q, k, v, g, vaug = pl.pallas_call(
        _proj_body,
        grid=(NQ_,),
        in_specs=[
            pl.BlockSpec((QB_, D_), lambda i: (i, 0)),
            pl.BlockSpec((D_, PPAD_), lambda i: (0, 0)),
        ],
        out_specs=[
            pl.BlockSpec((H_, QB_, HD_), lambda i: (0, i, 0)),
            pl.BlockSpec((QB_, 512), lambda i: (i, 0)),
            pl.BlockSpec((QB_, 512), lambda i: (i, 0)),
            pl.BlockSpec((QB_, 128), lambda i: (i, 0)),
            pl.BlockSpec((QB_, 1024), lambda i: (i, 0)),
        ],
        out_shape=[
            jax.ShapeDtypeStruct((H_, T_, HD_), f32),
            jax.ShapeDtypeStruct((T_, 512), f32),
            jax.ShapeDtypeStruct((T_, 512), f32),
            jax.ShapeDtypeStruct((T_, 128), f32),
            jax.ShapeDtypeStruct((T_, 1024), jnp.bfloat16),
        ],
        interpret=interpret,
    )(x, WcatT)

    kb, vb = pl.pallas_call(
        _pool_body,
        out_shape=[
            jax.ShapeDtypeStruct((NB_, HKV_ * HD_), f32),
            jax.ShapeDtypeStruct((NB_, HKV_ * HD_), f32),
        ],
        interpret=interpret,
    )(k, v)

    ocmp, sel = pl.pallas_call(
        _cmp_body,
        grid=(NQ_,),
        in_specs=[
            pl.BlockSpec((H_, QB_, HD_), lambda i: (0, i, 0)),
            pl.BlockSpec((NB_, HKV_ * HD_), lambda i: (0, 0)),
            pl.BlockSpec((NB_, HKV_ * HD_), lambda i: (0, 0)),
        ],
        out_specs=[
            pl.BlockSpec((QB_, 2048), lambda i: (i, 0)),
            pl.BlockSpec((QB_, HKV_ * NB_), lambda i: (i, 0)),
        ],
        out_shape=[
            jax.ShapeDtypeStruct((T_, 2048), f32),
            jax.ShapeDtypeStruct((T_, HKV_ * NB_), f32),
        ],
        interpret=interpret,
    )(q, kb, vb)

    z = pl.pallas_call(
        _attn_body,
        grid=(NQ_,),
        in_specs=[
            pl.BlockSpec((H_, QB_, HD_), lambda i: (0, i, 0)),
            pl.BlockSpec((T_, 512), lambda i: (0, 0)),
            pl.BlockSpec((QB_, 128), lambda i: (i, 0)),
            pl.BlockSpec((QB_, 2048), lambda i: (i, 0)),
            pl.BlockSpec((QB_, HKV_ * NB_), lambda i: (i, 0)),
            pl.BlockSpec((T_, 1024), lambda i: (0, 0)),
        ],
        out_specs=pl.BlockSpec((QB_, 2048), lambda i: (i, 0)),
        out_shape=jax.ShapeDtypeStruct((T_, 2048), f32),
        interpret=interpret,
    )(q, k, g, ocmp, sel, vaug)

    out = pl.pallas_call(
        _oproj_body,
        grid=(NQ_,),
        in_specs=[
            pl.BlockSpec((QB_, 2048), lambda i: (i, 0)),
            pl.BlockSpec((D_, D_), lambda i: (0, 0)),
        ],
        out_specs=pl.BlockSpec((QB_, D_), lambda i: (i, 0)),
        out_shape=jax.ShapeDtypeStruct((T_, D_), f32),
        interpret=interpret,
    )(z, WoT)
    return out


def kernel(hidden_states, Wq, Wk, Wv, Wg, Wo):
    x = hidden_states.reshape(T_, D_)
    Wcat = jnp.concatenate([Wq, Wk, Wv,
                            jnp.pad(Wg, ((0, PPAD_ - 3072 - H_ * 3), (0, 0)))], axis=0)
    out = _nsa_pallas(x, Wcat.T, Wo.T)
    return out.reshape(B_, T_, D_)


# speedup vs baseline: 2.7332x; 1.0093x over previous
import jax
import jax.numpy as jnp
from jax.experimental import pallas as pl
from jax.experimental.pallas import tpu as pltpu

B_, T_, D_ = 1, 2048, 2048
H_, HKV_, HD_ = 16, 4, 128
BS_, K_, W_ = 64, 16, 512
NB_ = T_ // BS_
G_ = H_ // HKV_
NEG_ = -1e30
SCALE_ = HD_ ** -0.5
QB_ = 256
NQ_ = T_ // QB_
PPAD_ = 3200


def _msoftmax(s, mask):
    s = jnp.where(mask, s, NEG_)
    m = jnp.max(s, axis=-1, keepdims=True)
    e = jnp.where(mask, jnp.exp(s - m), 0.0)
    d = jnp.sum(e, axis=-1, keepdims=True)
    return e / jnp.maximum(d, 1e-20)


def _proj_body(x_ref, w_ref, q_ref, k_ref, v_ref, g_ref, vaug_ref):
    y = jax.lax.dot_general(x_ref[...], w_ref[...], (((1,), (0,)), ((), ())),
                            preferred_element_type=jnp.float32)
    for h in range(H_):
        q_ref[h] = y[:, h * HD_:(h + 1) * HD_] * SCALE_
    k_ref[...] = y[:, 2048:2560]
    v_ref[...] = y[:, 2560:3072]
    g_ref[...] = y[:, 3072:3200]
    for hk in range(HKV_):
        vaug_ref[:, hk * 256:hk * 256 + HD_] = (
            y[:, 2560 + hk * HD_:2560 + (hk + 1) * HD_].astype(jnp.bfloat16))
        vaug_ref[:, hk * 256 + HD_:(hk + 1) * 256] = jnp.ones(
            (QB_, 128), jnp.bfloat16)


def _pool_body(k_ref, v_ref, kb_ref, vb_ref):
    kb_ref[...] = jnp.mean(k_ref[...].reshape(NB_, BS_, HKV_ * HD_), axis=1)
    vb_ref[...] = jnp.mean(v_ref[...].reshape(NB_, BS_, HKV_ * HD_), axis=1)


def _cmp_body(q_ref, kb_ref, vb_ref, ocmp_ref, sel_ref):
    i = pl.program_id(0)
    nr = jax.lax.broadcasted_iota(jnp.int32, (NB_, G_ * QB_), 0)
    rc = jax.lax.broadcasted_iota(jnp.int32, (NB_, G_ * QB_), 1)
    t4 = i * QB_ + rc % QB_
    m_cmpT = ((nr + 1) * BS_ - 1) <= t4
    n1 = jax.lax.broadcasted_iota(jnp.int32, (NB_, QB_), 0)
    tl = i * QB_ + jax.lax.broadcasted_iota(jnp.int32, (NB_, QB_), 1)
    forceT = (n1 == (tl // BS_)) | (n1 == 0)
    force_add = jnp.where(forceT, 1e9, 0.0)
    for hk in range(HKV_):
        kb = kb_ref[:, hk * HD_:(hk + 1) * HD_]
        vb = vb_ref[:, hk * HD_:(hk + 1) * HD_]
        qg = q_ref[hk * G_:(hk + 1) * G_].reshape(G_ * QB_, HD_)
        sT = jax.lax.dot_general(kb, qg, (((1,), (1,)), ((), ())),
                                 preferred_element_type=jnp.float32)
        e = jnp.exp(jnp.where(m_cmpT, sT, NEG_))
        d = jnp.sum(e, axis=0, keepdims=True)
        pT = e / jnp.maximum(d, 1e-20)
        o = jax.lax.dot_general(pT, vb, (((0,), (0,)), ((), ())),
                                preferred_element_type=jnp.float32)
        for g in range(G_):
            h = hk * G_ + g
            ocmp_ref[:, h * HD_:(h + 1) * HD_] = o[g * QB_:(g + 1) * QB_]
        impT = jnp.sum(pT.reshape(NB_, G_, QB_), axis=1) + force_add
        cnt = jnp.zeros((NB_, QB_), jnp.float32)
        for mrow in range(NB_):
            vm = impT[mrow:mrow + 1, :]
            beats = (vm > impT) | ((vm == impT) & (mrow < n1))
            cnt = cnt + beats.astype(jnp.float32)
        selT = (cnt < K_).astype(jnp.float32)
        sel_ref[:, hk * NB_:(hk + 1) * NB_] = selT.T


def _make_attn_body(i):
    Ti = (i + 1) * QB_
    j0 = max(0, i - 2)
    c0 = j0 * QB_
    Wi = Ti - c0

    def _attn_body(q_ref, k_ref, g_ref, ocmp_ref, sel_ref, vaug_ref, o_ref):
        gates = jax.nn.sigmoid(g_ref[:, :H_ * 3])
        r4 = jax.lax.broadcasted_iota(jnp.int32, (G_ * QB_, Ti), 0)
        t4 = i * QB_ + r4 % QB_
        c4 = jax.lax.broadcasted_iota(jnp.int32, (G_ * QB_, Ti), 1)
        causal4 = c4 <= t4
        rw = jax.lax.broadcasted_iota(jnp.int32, (G_ * QB_, Wi), 0)
        cw = c0 + jax.lax.broadcasted_iota(jnp.int32, (G_ * QB_, Wi), 1)
        swa4 = cw > (i * QB_ + rw % QB_) - W_
        en = jax.lax.broadcasted_iota(jnp.int32, (NB_, Ti), 0)
        es_ = jax.lax.broadcasted_iota(jnp.int32, (NB_, Ti), 1)
        expand = ((es_ // BS_) == en).astype(jnp.float32)
        for hk in range(HKV_):
            kk = k_ref[:, hk * HD_:(hk + 1) * HD_]
            vvaug = vaug_ref[:, hk * 256:(hk + 1) * 256]
            selc = sel_ref[:, hk * NB_:(hk + 1) * NB_]
            selx = jax.lax.dot_general(selc, expand, (((1,), (0,)), ((), ())),
                                       preferred_element_type=jnp.float32)
            sel4 = jnp.broadcast_to((selx > 0.5)[None], (G_, QB_, Ti)).reshape(
                G_ * QB_, Ti)
            qg = q_ref[hk * G_:(hk + 1) * G_].reshape(G_ * QB_, HD_)
            s = jax.lax.dot_general(qg, kk, (((1,), (1,)), ((), ())),
                                    preferred_element_type=jnp.float32)
            es = jnp.exp(jnp.where(causal4, s, NEG_))
            e_slc = jnp.where(sel4, es, 0.0).astype(jnp.bfloat16)
            e_swa = jnp.where(swa4, es[:, c0:], 0.0).astype(jnp.bfloat16)
            nd_slc = jax.lax.dot_general(e_slc, vvaug, (((1,), (0,)), ((), ())),
                                         preferred_element_type=jnp.float32)
            nd_swa = jax.lax.dot_general(e_swa, vvaug[c0:], (((1,), (0,)), ((), ())),
                                         preferred_element_type=jnp.float32)
            o_slc = nd_slc[:, :HD_] / jnp.maximum(nd_slc[:, HD_:HD_ + 1], 1e-20)
            o_swa = nd_swa[:, :HD_] / jnp.maximum(nd_swa[:, HD_:HD_ + 1], 1e-20)
            for g in range(G_):
                h = hk * G_ + g
                rows = slice(g * QB_, (g + 1) * QB_)
                cols = slice(h * HD_, (h + 1) * HD_)
                gc = gates[:, 3 * h:3 * h + 1]
                gs = gates[:, 3 * h + 1:3 * h + 2]
                gw = gates[:, 3 * h + 2:3 * h + 3]
                o_ref[:, cols] = (ocmp_ref[:, cols] * gc + o_slc[rows] * gs
                                  + o_swa[rows] * gw)

    return _attn_body


def _oproj_body(z_ref, w_ref, o_ref):
    o_ref[...] = jax.lax.dot_general(z_ref[...], w_ref[...], (((1,), (0,)), ((), ())),
                                     preferred_element_type=jnp.float32)


def _nsa_pallas(x, WcatT, WoT, interpret=False):
    f32 = jnp.float32
    q, k, v, g, vaug = pl.pallas_call(
        _proj_body,
        grid=(NQ_,),
        in_specs=[
            pl.BlockSpec((QB_, D_), lambda i: (i, 0)),
            pl.BlockSpec((D_, PPAD_), lambda i: (0, 0)),
        ],
        out_specs=[
            pl.BlockSpec((H_, QB_, HD_), lambda i: (0, i, 0)),
            pl.BlockSpec((QB_, 512), lambda i: (i, 0)),
            pl.BlockSpec((QB_, 512), lambda i: (i, 0)),
            pl.BlockSpec((QB_, 128), lambda i: (i, 0)),
            pl.BlockSpec((QB_, 1024), lambda i: (i, 0)),
        ],
        out_shape=[
            jax.ShapeDtypeStruct((H_, T_, HD_), f32),
            jax.ShapeDtypeStruct((T_, 512), f32),
            jax.ShapeDtypeStruct((T_, 512), f32),
            jax.ShapeDtypeStruct((T_, 128), f32),
            jax.ShapeDtypeStruct((T_, 1024), jnp.bfloat16),
        ],
        interpret=interpret,
    )(x, WcatT)

    kb, vb = pl.pallas_call(
        _pool_body,
        out_shape=[
            jax.ShapeDtypeStruct((NB_, HKV_ * HD_), f32),
            jax.ShapeDtypeStruct((NB_, HKV_ * HD_), f32),
        ],
        interpret=interpret,
    )(k, v)

    ocmp, sel = pl.pallas_call(
        _cmp_body,
        grid=(NQ_,),
        in_specs=[
            pl.BlockSpec((H_, QB_, HD_), lambda i: (0, i, 0)),
            pl.BlockSpec((NB_, HKV_ * HD_), lambda i: (0, 0)),
            pl.BlockSpec((NB_, HKV_ * HD_), lambda i: (0, 0)),
        ],
        out_specs=[
            pl.BlockSpec((QB_, 2048), lambda i: (i, 0)),
            pl.BlockSpec((QB_, HKV_ * NB_), lambda i: (i, 0)),
        ],
        out_shape=[
            jax.ShapeDtypeStruct((T_, 2048), f32),
            jax.ShapeDtypeStruct((T_, HKV_ * NB_), f32),
        ],
        interpret=interpret,
    )(q, kb, vb)

    zs = []
    for i in range(NQ_):
        Ti = (i + 1) * QB_
        zi = pl.pallas_call(
            _make_attn_body(i),
            grid=(1,),
            in_specs=[
                pl.BlockSpec((H_, QB_, HD_), lambda _, i=i: (0, i, 0)),
                pl.BlockSpec((Ti, 512), lambda _: (0, 0)),
                pl.BlockSpec((QB_, 128), lambda _, i=i: (i, 0)),
                pl.BlockSpec((QB_, 2048), lambda _, i=i: (i, 0)),
                pl.BlockSpec((QB_, HKV_ * NB_), lambda _, i=i: (i, 0)),
                pl.BlockSpec((Ti, 1024), lambda _: (0, 0)),
            ],
            out_specs=pl.BlockSpec((QB_, 2048), lambda _: (0, 0)),
            out_shape=jax.ShapeDtypeStruct((QB_, 2048), f32),
            interpret=interpret,
        )(q, k, g, ocmp, sel, vaug)
        zs.append(zi)
    z = jnp.concatenate(zs, axis=0)

    out = pl.pallas_call(
        _oproj_body,
        grid=(NQ_,),
        in_specs=[
            pl.BlockSpec((QB_, 2048), lambda i: (i, 0)),
            pl.BlockSpec((D_, D_), lambda i: (0, 0)),
        ],
        out_specs=pl.BlockSpec((QB_, D_), lambda i: (i, 0)),
        out_shape=jax.ShapeDtypeStruct((T_, D_), f32),
        interpret=interpret,
    )(z, WoT)
    return out


def kernel(hidden_states, Wq, Wk, Wv, Wg, Wo):
    x = hidden_states.reshape(T_, D_)
    Wcat = jnp.concatenate([Wq, Wk, Wv,
                            jnp.pad(Wg, ((0, PPAD_ - 3072 - H_ * 3), (0, 0)))], axis=0)
    out = _nsa_pallas(x, Wcat.T, Wo.T)
    return out.reshape(B_, T_, D_)


# speedup vs baseline: 3.1083x; 1.1373x over previous
import jax
import jax.numpy as jnp
from jax.experimental import pallas as pl

B_, T_, D_ = 1, 2048, 2048
H_, HKV_, HD_ = 16, 4, 128
BS_, K_, W_ = 64, 16, 512
NB_ = T_ // BS_
G_ = H_ // HKV_
NEG_ = -1e30
SCALE_ = HD_ ** -0.5
QB_ = 256
NQ_ = T_ // QB_
PPAD_ = 3200
BF_ = jnp.bfloat16


def _proj_body(x_ref, w_ref, q_ref, k_ref, kf_ref, g_ref, vaug_ref):
    y = jax.lax.dot_general(x_ref[...], w_ref[...], (((1,), (0,)), ((), ())),
                            preferred_element_type=jnp.float32)
    for h in range(H_):
        q_ref[h] = (y[:, h * HD_:(h + 1) * HD_] * SCALE_).astype(BF_)
    k_ref[...] = y[:, 2048:2560].astype(BF_)
    kf_ref[...] = y[:, 2048:2560]
    g_ref[...] = y[:, 3072:3200]
    for hk in range(HKV_):
        vaug_ref[:, hk * 256:hk * 256 + HD_] = (
            y[:, 2560 + hk * HD_:2560 + (hk + 1) * HD_].astype(BF_))
        vaug_ref[:, hk * 256 + HD_:(hk + 1) * 256] = jnp.ones(
            (QB_, 128), BF_)


def _pool_body(kf_ref, vaug_ref, kb_ref, vb_ref):
    kb_ref[...] = jnp.mean(kf_ref[...].reshape(
        NB_, BS_, HKV_ * HD_), axis=1).astype(BF_)
    va = vaug_ref[...].astype(jnp.float32).reshape(NB_, BS_, HKV_ * 256)
    vm = jnp.mean(va, axis=1)
    for hk in range(HKV_):
        vb_ref[:, hk * HD_:(hk + 1) * HD_] = vm[:, hk * 256:hk * 256 + HD_]


def _make_attn_body(i):
    Ti = (i + 1) * QB_
    j0 = max(0, i - 2)
    c0 = j0 * QB_
    Wi = Ti - c0

    def _attn_body(q_ref, k_ref, g_ref, kb_ref, vb_ref, vaug_ref, o_ref):
        gates = jax.nn.sigmoid(g_ref[:, :H_ * 3])
        r4 = jax.lax.broadcasted_iota(jnp.int32, (G_ * QB_, Ti), 0)
        t4 = i * QB_ + r4 % QB_
        c4 = jax.lax.broadcasted_iota(jnp.int32, (G_ * QB_, Ti), 1)
        causal4 = c4 <= t4
        rw = jax.lax.broadcasted_iota(jnp.int32, (G_ * QB_, Wi), 0)
        cw = c0 + jax.lax.broadcasted_iota(jnp.int32, (G_ * QB_, Wi), 1)
        swa4 = cw > (i * QB_ + rw % QB_) - W_
        en = jax.lax.broadcasted_iota(jnp.int32, (NB_, Ti), 0)
        es_ = jax.lax.broadcasted_iota(jnp.int32, (NB_, Ti), 1)
        expand = ((es_ // BS_) == en).astype(jnp.float32)
        nr = jax.lax.broadcasted_iota(jnp.int32, (NB_, G_ * QB_), 0)
        rc = jax.lax.broadcasted_iota(jnp.int32, (NB_, G_ * QB_), 1)
        tt4 = i * QB_ + rc % QB_
        m_cmpT = ((nr + 1) * BS_ - 1) <= tt4
        n1 = jax.lax.broadcasted_iota(jnp.int32, (NB_, QB_), 0)
        tl = i * QB_ + jax.lax.broadcasted_iota(jnp.int32, (NB_, QB_), 1)
        forceT = (n1 == (tl // BS_)) | (n1 == 0)
        force_add = jnp.where(forceT, 1e9, 0.0)
        for hk in range(HKV_):
            kb = kb_ref[:, hk * HD_:(hk + 1) * HD_]
            vb = vb_ref[:, hk * HD_:(hk + 1) * HD_]
            qg = q_ref[hk * G_:(hk + 1) * G_].reshape(G_ * QB_, HD_)
            sT = jax.lax.dot_general(kb, qg, (((1,), (1,)), ((), ())),
                                     preferred_element_type=jnp.float32)
            ec = jnp.exp(jnp.where(m_cmpT, sT, NEG_))
            dc = jnp.sum(ec, axis=0, keepdims=True)
            pT = ec / jnp.maximum(dc, 1e-20)
            ocmp = jax.lax.dot_general(pT, vb, (((0,), (0,)), ((), ())),
                                       preferred_element_type=jnp.float32)
            impT = jnp.sum(pT.reshape(NB_, G_, QB_), axis=1) + force_add
            cnt = jnp.zeros((NB_, QB_), jnp.float32)
            for mrow in range(NB_):
                vm = impT[mrow:mrow + 1, :]
                beats = (vm > impT) | ((vm == impT) & (mrow < n1))
                cnt = cnt + beats.astype(jnp.float32)
            selc = (cnt < K_).astype(jnp.float32).T
            kk = k_ref[:, hk * HD_:(hk + 1) * HD_]
            vvaug = vaug_ref[:, hk * 256:(hk + 1) * 256]
            selx = jax.lax.dot_general(selc, expand, (((1,), (0,)), ((), ())),
                                       preferred_element_type=jnp.float32)
            sel4 = jnp.broadcast_to((selx > 0.5)[None], (G_, QB_, Ti)).reshape(
                G_ * QB_, Ti)
            s = jax.lax.dot_general(qg, kk, (((1,), (1,)), ((), ())),
                                    preferred_element_type=jnp.float32)
            es = jnp.exp(jnp.where(causal4, s, NEG_))
            e_slc = jnp.where(sel4, es, 0.0).astype(BF_)
            e_swa = jnp.where(swa4, es[:, c0:], 0.0).astype(BF_)
            nd_slc = jax.lax.dot_general(e_slc, vvaug, (((1,), (0,)), ((), ())),
                                         preferred_element_type=jnp.float32)
            nd_swa = jax.lax.dot_general(e_swa, vvaug[c0:], (((1,), (0,)), ((), ())),
                                         preferred_element_type=jnp.float32)
            o_slc = nd_slc[:, :HD_] / jnp.maximum(nd_slc[:, HD_:HD_ + 1], 1e-20)
            o_swa = nd_swa[:, :HD_] / jnp.maximum(nd_swa[:, HD_:HD_ + 1], 1e-20)
            for g in range(G_):
                h = hk * G_ + g
                rows = slice(g * QB_, (g + 1) * QB_)
                cols = slice(h * HD_, (h + 1) * HD_)
                gc = gates[:, 3 * h:3 * h + 1]
                gs = gates[:, 3 * h + 1:3 * h + 2]
                gw = gates[:, 3 * h + 2:3 * h + 3]
                o_ref[:, cols] = (ocmp[rows] * gc + o_slc[rows] * gs
                                  + o_swa[rows] * gw).astype(BF_)

    return _attn_body


def _oproj_body(z_ref, w_ref, o_ref):
    o_ref[...] = jax.lax.dot_general(z_ref[...], w_ref[...], (((1,), (0,)), ((), ())),
                                     preferred_element_type=jnp.float32)


def _nsa_pallas(x, WcatT, WoT, interpret=False):
    f32 = jnp.float32
    q, k, kf, g, vaug = pl.pallas_call(
        _proj_body,
        grid=(NQ_,),
        in_specs=[
            pl.BlockSpec((QB_, D_), lambda i: (i, 0)),
            pl.BlockSpec((D_, PPAD_), lambda i: (0, 0)),
        ],
        out_specs=[
            pl.BlockSpec((H_, QB_, HD_), lambda i: (0, i, 0)),
            pl.BlockSpec((QB_, 512), lambda i: (i, 0)),
            pl.BlockSpec((QB_, 512), lambda i: (i, 0)),
            pl.BlockSpec((QB_, 128), lambda i: (i, 0)),
            pl.BlockSpec((QB_, 1024), lambda i: (i, 0)),
        ],
        out_shape=[
            jax.ShapeDtypeStruct((H_, T_, HD_), BF_),
            jax.ShapeDtypeStruct((T_, 512), BF_),
            jax.ShapeDtypeStruct((T_, 512), f32),
            jax.ShapeDtypeStruct((T_, 128), f32),
            jax.ShapeDtypeStruct((T_, 1024), BF_),
        ],
        interpret=interpret,
    )(x, WcatT)

    kb, vb = pl.pallas_call(
        _pool_body,
        out_shape=[
            jax.ShapeDtypeStruct((NB_, HKV_ * HD_), BF_),
            jax.ShapeDtypeStruct((NB_, HKV_ * HD_), f32),
        ],
        interpret=interpret,
    )(kf, vaug)

    zs = []
    for i in range(NQ_):
        Ti = (i + 1) * QB_
        zi = pl.pallas_call(
            _make_attn_body(i),
            grid=(1,),
            in_specs=[
                pl.BlockSpec((H_, QB_, HD_), lambda _, i=i: (0, i, 0)),
                pl.BlockSpec((Ti, 512), lambda _: (0, 0)),
                pl.BlockSpec((QB_, 128), lambda _, i=i: (i, 0)),
                pl.BlockSpec((NB_, HKV_ * HD_), lambda _: (0, 0)),
                pl.BlockSpec((NB_, HKV_ * HD_), lambda _: (0, 0)),
                pl.BlockSpec((Ti, 1024), lambda _: (0, 0)),
            ],
            out_specs=pl.BlockSpec((QB_, 2048), lambda _: (0, 0)),
            out_shape=jax.ShapeDtypeStruct((QB_, 2048), BF_),
            interpret=interpret,
        )(q, k, g, kb, vb, vaug)
        zs.append(zi)
    z = jnp.concatenate(zs, axis=0)

    out = pl.pallas_call(
        _oproj_body,
        grid=(NQ_,),
        in_specs=[
            pl.BlockSpec((QB_, 2048), lambda i: (i, 0)),
            pl.BlockSpec((D_, D_), lambda i: (0, 0)),
        ],
        out_specs=pl.BlockSpec((QB_, D_), lambda i: (i, 0)),
        out_shape=jax.ShapeDtypeStruct((T_, D_), f32),
        interpret=interpret,
    )(z, WoT)
    return out


def kernel(hidden_states, Wq, Wk, Wv, Wg, Wo):
    x = hidden_states.reshape(T_, D_).astype(BF_)
    Wcat = jnp.concatenate([Wq, Wk, Wv,
                            jnp.pad(Wg, ((0, PPAD_ - 3072 - H_ * 3), (0, 0)))], axis=0)
    out = _nsa_pallas(x, Wcat.T.astype(BF_), Wo.T.astype(BF_))
    return out.reshape(B_, T_, D_)


# speedup vs baseline: 3.1851x; 1.0247x over previous
import jax
import jax.numpy as jnp
from jax.experimental import pallas as pl

B_, T_, D_ = 1, 2048, 2048
H_, HKV_, HD_ = 16, 4, 128
BS_, K_, W_ = 64, 16, 512
NB_ = T_ // BS_
G_ = H_ // HKV_
NEG_ = -1e30
SCALE_ = HD_ ** -0.5
QB_ = 256
NQ_ = T_ // QB_
PPAD_ = 3200
BF_ = jnp.bfloat16


def _proj_body(x_ref, w_ref, q_ref, k_ref, g_ref, vaug_ref, kb_ref, vb_ref):
    y = jax.lax.dot_general(x_ref[...], w_ref[...], (((1,), (0,)), ((), ())),
                            preferred_element_type=jnp.float32)
    for h in range(H_):
        q_ref[h] = (y[:, h * HD_:(h + 1) * HD_] * SCALE_).astype(BF_)
    k_ref[...] = y[:, 2048:2560].astype(BF_)
    g_ref[...] = y[:, 3072:3200]
    kb_ref[0] = jnp.mean(y[:, 2048:2560].reshape(
        QB_ // BS_, BS_, HKV_ * HD_), axis=1).astype(BF_)
    vb_ref[0] = jnp.mean(y[:, 2560:3072].reshape(
        QB_ // BS_, BS_, HKV_ * HD_), axis=1)
    for hk in range(HKV_):
        vaug_ref[:, hk * 256:hk * 256 + HD_] = (
            y[:, 2560 + hk * HD_:2560 + (hk + 1) * HD_].astype(BF_))
        vaug_ref[:, hk * 256 + HD_:(hk + 1) * 256] = jnp.ones(
            (QB_, 128), BF_)


def _make_attn_body(i):
    Ti = (i + 1) * QB_
    j0 = max(0, i - 2)
    c0 = j0 * QB_
    Wi = Ti - c0

    def _attn_body(q_ref, k_ref, g_ref, kb_ref, vb_ref, vaug_ref, o_ref):
        kb_full = kb_ref[...].reshape(NB_, HKV_ * HD_)
        vb_full = vb_ref[...].reshape(NB_, HKV_ * HD_)
        gates = jax.nn.sigmoid(g_ref[:, :H_ * 3])
        r4 = jax.lax.broadcasted_iota(jnp.int32, (G_ * QB_, Ti), 0)
        t4 = i * QB_ + r4 % QB_
        c4 = jax.lax.broadcasted_iota(jnp.int32, (G_ * QB_, Ti), 1)
        causal4 = c4 <= t4
        rw = jax.lax.broadcasted_iota(jnp.int32, (G_ * QB_, Wi), 0)
        cw = c0 + jax.lax.broadcasted_iota(jnp.int32, (G_ * QB_, Wi), 1)
        swa4 = cw > (i * QB_ + rw % QB_) - W_
        en = jax.lax.broadcasted_iota(jnp.int32, (NB_, Ti), 0)
        es_ = jax.lax.broadcasted_iota(jnp.int32, (NB_, Ti), 1)
        expand = ((es_ // BS_) == en).astype(jnp.float32)
        nr = jax.lax.broadcasted_iota(jnp.int32, (NB_, G_ * QB_), 0)
        rc = jax.lax.broadcasted_iota(jnp.int32, (NB_, G_ * QB_), 1)
        tt4 = i * QB_ + rc % QB_
        m_cmpT = ((nr + 1) * BS_ - 1) <= tt4
        n1 = jax.lax.broadcasted_iota(jnp.int32, (NB_, QB_), 0)
        tl = i * QB_ + jax.lax.broadcasted_iota(jnp.int32, (NB_, QB_), 1)
        forceT = (n1 == (tl // BS_)) | (n1 == 0)
        force_add = jnp.where(forceT, 1e9, 0.0)
        for hk in range(HKV_):
            kb = kb_full[:, hk * HD_:(hk + 1) * HD_]
            vb = vb_full[:, hk * HD_:(hk + 1) * HD_]
            qg = q_ref[hk * G_:(hk + 1) * G_].reshape(G_ * QB_, HD_)
            sT = jax.lax.dot_general(kb, qg, (((1,), (1,)), ((), ())),
                                     preferred_element_type=jnp.float32)
            ec = jnp.exp(jnp.where(m_cmpT, sT, NEG_))
            dc = jnp.sum(ec, axis=0, keepdims=True)
            pT = ec / jnp.maximum(dc, 1e-20)
            ocmp = jax.lax.dot_general(pT, vb, (((0,), (0,)), ((), ())),
                                       preferred_element_type=jnp.float32)
            impT = jnp.sum(pT.reshape(NB_, G_, QB_), axis=1) + force_add
            cnt = jnp.zeros((NB_, QB_), jnp.float32)
            for mrow in range(NB_):
                vm = impT[mrow:mrow + 1, :]
                beats = (vm > impT) | ((vm == impT) & (mrow < n1))
                cnt = cnt + beats.astype(jnp.float32)
            selc = (cnt < K_).astype(jnp.float32).T
            kk = k_ref[:, hk * HD_:(hk + 1) * HD_]
            vvaug = vaug_ref[:, hk * 256:(hk + 1) * 256]
            selx = jax.lax.dot_general(selc, expand, (((1,), (0,)), ((), ())),
                                       preferred_element_type=jnp.float32)
            sel4 = jnp.broadcast_to((selx > 0.5)[None], (G_, QB_, Ti)).reshape(
                G_ * QB_, Ti)
            s = jax.lax.dot_general(qg, kk, (((1,), (1,)), ((), ())),
                                    preferred_element_type=jnp.float32)
            es = jnp.exp(jnp.where(causal4, s, NEG_))
            e_slc = jnp.where(sel4, es, 0.0).astype(BF_)
            e_swa = jnp.where(swa4, es[:, c0:], 0.0).astype(BF_)
            nd_slc = jax.lax.dot_general(e_slc, vvaug, (((1,), (0,)), ((), ())),
                                         preferred_element_type=jnp.float32)
            nd_swa = jax.lax.dot_general(e_swa, vvaug[c0:], (((1,), (0,)), ((), ())),
                                         preferred_element_type=jnp.float32)
            o_slc = nd_slc[:, :HD_] / jnp.maximum(nd_slc[:, HD_:HD_ + 1], 1e-20)
            o_swa = nd_swa[:, :HD_] / jnp.maximum(nd_swa[:, HD_:HD_ + 1], 1e-20)
            for g in range(G_):
                h = hk * G_ + g
                rows = slice(g * QB_, (g + 1) * QB_)
                cols = slice(h * HD_, (h + 1) * HD_)
                gc = gates[:, 3 * h:3 * h + 1]
                gs = gates[:, 3 * h + 1:3 * h + 2]
                gw = gates[:, 3 * h + 2:3 * h + 3]
                o_ref[:, cols] = (ocmp[rows] * gc + o_slc[rows] * gs
                                  + o_swa[rows] * gw).astype(BF_)

    return _attn_body


def _oproj_body(z_ref, w_ref, o_ref):
    o_ref[...] = jax.lax.dot_general(z_ref[...], w_ref[...], (((1,), (0,)), ((), ())),
                                     preferred_element_type=jnp.float32)


def _nsa_pallas(x, WcatT, WoT, interpret=False):
    f32 = jnp.float32
    q, k, g, vaug, kb, vb = pl.pallas_call(
        _proj_body,
        grid=(NQ_,),
        in_specs=[
            pl.BlockSpec((QB_, D_), lambda i: (i, 0)),
            pl.BlockSpec((D_, PPAD_), lambda i: (0, 0)),
        ],
        out_specs=[
            pl.BlockSpec((H_, QB_, HD_), lambda i: (0, i, 0)),
            pl.BlockSpec((QB_, 512), lambda i: (i, 0)),
            pl.BlockSpec((QB_, 128), lambda i: (i, 0)),
            pl.BlockSpec((QB_, 1024), lambda i: (i, 0)),
            pl.BlockSpec((1, QB_ // BS_, 512), lambda i: (i, 0, 0)),
            pl.BlockSpec((1, QB_ // BS_, 512), lambda i: (i, 0, 0)),
        ],
        out_shape=[
            jax.ShapeDtypeStruct((H_, T_, HD_), BF_),
            jax.ShapeDtypeStruct((T_, 512), BF_),
            jax.ShapeDtypeStruct((T_, 128), f32),
            jax.ShapeDtypeStruct((T_, 1024), BF_),
            jax.ShapeDtypeStruct((NQ_, QB_ // BS_, 512), BF_),
            jax.ShapeDtypeStruct((NQ_, QB_ // BS_, 512), f32),
        ],
        interpret=interpret,
    )(x, WcatT)

    zs = []
    for i in range(NQ_):
        Ti = (i + 1) * QB_
        zi = pl.pallas_call(
            _make_attn_body(i),
            grid=(1,),
            in_specs=[
                pl.BlockSpec((H_, QB_, HD_), lambda _, i=i: (0, i, 0)),
                pl.BlockSpec((Ti, 512), lambda _: (0, 0)),
                pl.BlockSpec((QB_, 128), lambda _, i=i: (i, 0)),
                pl.BlockSpec((NQ_, QB_ // BS_, 512), lambda _: (0, 0, 0)),
                pl.BlockSpec((NQ_, QB_ // BS_, 512), lambda _: (0, 0, 0)),
                pl.BlockSpec((Ti, 1024), lambda _: (0, 0)),
            ],
            out_specs=pl.BlockSpec((QB_, 2048), lambda _: (0, 0)),
            out_shape=jax.ShapeDtypeStruct((QB_, 2048), BF_),
            interpret=interpret,
        )(q, k, g, kb, vb, vaug)
        zs.append(zi)
    z = jnp.concatenate(zs, axis=0)

    out = pl.pallas_call(
        _oproj_body,
        grid=(NQ_,),
        in_specs=[
            pl.BlockSpec((QB_, 2048), lambda i: (i, 0)),
            pl.BlockSpec((D_, D_), lambda i: (0, 0)),
        ],
        out_specs=pl.BlockSpec((QB_, D_), lambda i: (i, 0)),
        out_shape=jax.ShapeDtypeStruct((T_, D_), f32),
        interpret=interpret,
    )(z, WoT)
    return out


def kernel(hidden_states, Wq, Wk, Wv, Wg, Wo):
    x = hidden_states.reshape(T_, D_).astype(BF_)
    Wcat = jnp.concatenate([Wq, Wk, Wv,
                            jnp.pad(Wg, ((0, PPAD_ - 3072 - H_ * 3), (0, 0)))], axis=0)
    out = _nsa_pallas(x, Wcat.T.astype(BF_), Wo.T.astype(BF_))
    return out.reshape(B_, T_, D_)
